# baseline (speedup 1.0000x reference)
import functools

import jax
import jax.numpy as jnp
from jax import lax
from jax.experimental import pallas as pl
from jax.experimental.pallas import tpu as pltpu
from jax.experimental.pallas import tpu_sc as plsc

N = 10000
E = 320000
D = 128
NC = 2
NS = 16
NW = NC * NS
CH = 128
RB = 1024
NP = 10240
RPT = NP // NS

SEG_CPW = 80
SEG_PAD = NW * SEG_CPW * CH
DEC_CPW = 160
DEC_ROWS = NW * DEC_CPW
DEC_PAD = DEC_ROWS * CH
DEC_OUT = (DEC_ROWS + 2 * NW) * CH

_mesh = plsc.VectorSubcoreMesh(
    core_axis_name="c", subcore_axis_name="s", num_cores=NC, num_subcores=NS)


def _enc_prep_body(x_ref, wl_ref, wr_ref, bl_ref, y_ref, r_ref):
    xb = x_ref[...]
    y_ref[...] = jnp.dot(xb, wl_ref[...].T, preferred_element_type=jnp.float32)
    r_ref[...] = (jnp.dot(xb, wr_ref[...].T, preferred_element_type=jnp.float32)
                  + bl_ref[...])


def _inv_count_col(cnt_blk):
    ones = jnp.ones((NW, 1), jnp.float32)
    col = lax.dot_general(cnt_blk, ones, (((0,), (0,)), ((), ())),
                          preferred_element_type=jnp.float32)
    return 1.0 / jnp.maximum(col, 1.0)


def _combine1_body(p_ref, cnt_ref, r1_ref, wl2_ref, wr2_ref, bl2_ref,
                   y2_ref, r2_ref, invb_ref):
    inv = _inv_count_col(cnt_ref[...])
    agg = p_ref[0] + p_ref[1]
    h = jnp.maximum(agg * inv + r1_ref[...], 0.0)
    y2_ref[...] = jnp.dot(h, wl2_ref[...].T, preferred_element_type=jnp.float32)
    r2_ref[...] = (jnp.dot(h, wr2_ref[...].T, preferred_element_type=jnp.float32)
                   + bl2_ref[...])
    invb_ref[...] = jnp.broadcast_to(inv, (inv.shape[0], D))


def _combine2_body(p_ref, r2_ref, invb_ref, w1a_ref, w1b_ref, bd1_ref,
                   a_ref, b_ref):
    agg = p_ref[0] + p_ref[1]
    z = agg * invb_ref[...] + r2_ref[...]
    a_ref[...] = jnp.dot(z, w1a_ref[...].T, preferred_element_type=jnp.float32)
    b_ref[...] = (jnp.dot(z, w1b_ref[...].T, preferred_element_type=jnp.float32)
                  + bd1_ref[...])


def _row_spec(width):
    return pl.BlockSpec((RB, width), lambda i: (i, 0))


def _full_spec(shape):
    return pl.BlockSpec(shape, lambda i: tuple(0 for _ in shape))


_enc_prep = pl.pallas_call(
    _enc_prep_body,
    grid=(pl.cdiv(N, RB),),
    in_specs=[_row_spec(D), _full_spec((D, D)), _full_spec((D, D)),
              _full_spec((1, D))],
    out_specs=[_row_spec(D), _row_spec(D)],
    out_shape=[jax.ShapeDtypeStruct((N, D), jnp.float32),
               jax.ShapeDtypeStruct((N, D), jnp.float32)],
)

_combine1 = pl.pallas_call(
    _combine1_body,
    grid=(pl.cdiv(N, RB),),
    in_specs=[pl.BlockSpec((NC, RB, D), lambda i: (0, i, 0)),
              pl.BlockSpec((NW, RB), lambda i: (0, i)),
              _row_spec(D), _full_spec((D, D)), _full_spec((D, D)),
              _full_spec((1, D))],
    out_specs=[_row_spec(D), _row_spec(D), _row_spec(D)],
    out_shape=[jax.ShapeDtypeStruct((N, D), jnp.float32),
               jax.ShapeDtypeStruct((N, D), jnp.float32),
               jax.ShapeDtypeStruct((N, D), jnp.float32)],
)

_combine2 = pl.pallas_call(
    _combine2_body,
    grid=(pl.cdiv(N, RB),),
    in_specs=[pl.BlockSpec((NC, RB, D), lambda i: (0, i, 0)),
              _row_spec(D), _row_spec(D), _full_spec((D, D)),
              _full_spec((D, D)), _full_spec((1, D))],
    out_specs=[_row_spec(D), _row_spec(D)],
    out_shape=[jax.ShapeDtypeStruct((N, D), jnp.float32),
               jax.ShapeDtypeStruct((N, D), jnp.float32)],
)


def _zero_spmem_slice(acc_sh, zrow_v, s):
    zr = zrow_v.shape[0]

    def zb(i, carry):
        for j in range(D // 16):
            zrow_v[i, pl.ds(j * 16, 16)] = jnp.zeros((16,), jnp.float32)
        return carry

    lax.fori_loop(0, zr, zb, 0)

    def zcp(k, carry):
        off = pl.multiple_of(s * RPT + k * zr, 8)
        pltpu.sync_copy(zrow_v, acc_sh.at[pl.ds(off, zr)])
        return carry

    lax.fori_loop(0, RPT // zr, zcp, 0)


def _make_segsum(with_count):
    def body(y_hbm, src_hbm, dst_hbm, *refs):
        if with_count:
            (agg_hbm, cnt_hbm, acc_sh, zrow_v,
             is0, is1, is2, is3, id0, id1, id2, id3,
             rows_v, hist_v, gsem, ism0, ism1, ism2, ism3) = refs
        else:
            (agg_hbm, acc_sh, zrow_v,
             is0, is1, is2, is3, id0, id1, id2, id3,
             rows_v, gsem, ism0, ism1, ism2, ism3) = refs
        isl = (is0, is1, is2, is3)
        idl = (id0, id1, id2, id3)
        isems = (ism0, ism1, ism2, ism3)
        c = lax.axis_index("c")
        s = lax.axis_index("s")
        wid = c * NS + s
        _zero_spmem_slice(acc_sh, zrow_v, s)
        if with_count:
            def zh(i, carry):
                hist_v[pl.ds(i * 16, 16)] = jnp.zeros((16,), jnp.float32)
                return carry

            lax.fori_loop(0, NP // 16, zh, 0)
        plsc.subcore_barrier()

        def ebase(g):
            return pl.multiple_of((wid * SEG_CPW + (g % SEG_CPW)) * CH, 8)

        def load_idx_sync(g, sl):
            pltpu.sync_copy(src_hbm.at[pl.ds(ebase(g), CH)], isl[sl])
            pltpu.sync_copy(dst_hbm.at[pl.ds(ebase(g), CH)], idl[sl])

        def load_idx(g, sl):
            pltpu.async_copy(src_hbm.at[pl.ds(ebase(g), CH)], isl[sl],
                             isems[sl])
            pltpu.async_copy(dst_hbm.at[pl.ds(ebase(g), CH)], idl[sl],
                             isems[sl])

        def wait_idx(sl):
            pltpu.make_async_copy(src_hbm.at[pl.ds(0, CH)], isl[sl],
                                  isems[sl]).wait()
            pltpu.make_async_copy(dst_hbm.at[pl.ds(0, CH)], idl[sl],
                                  isems[sl]).wait()

        def issue_gather(sl):
            pltpu.async_copy(y_hbm.at[isl[sl]], rows_v, gsem)

        def wait_gather():
            pltpu.make_async_copy(y_hbm.at[isl[0]], rows_v, gsem).wait()

        def hist(sl):
            for t in range(CH // 16):
                d16 = idl[sl][pl.ds(t * 16, 16)]
                cnts, lastm = plsc.scan_count(d16)
                plsc.addupdate_scatter(hist_v, [d16],
                                       cnts.astype(jnp.float32), mask=lastm)

        load_idx_sync(0, 0)
        issue_gather(0)
        load_idx(1, 1)
        load_idx(2, 2)
        load_idx(3, 3)

        def step(m, carry):
            for b in range(4):
                g = m * 4 + b
                sl = b
                wait_gather()
                if with_count:
                    hist(sl)
                pltpu.sync_copy(rows_v, acc_sh.at[idl[sl]], add=True)
                load_idx(g + 4, sl)
                wait_idx((sl + 1) % 4)
                issue_gather((sl + 1) % 4)
            return carry

        lax.fori_loop(0, SEG_CPW // 4, step, 0)
        wait_gather()
        wait_idx(1)
        wait_idx(2)
        wait_idx(3)

        if with_count:
            pltpu.sync_copy(hist_v, cnt_hbm.at[wid])
        plsc.subcore_barrier()
        off = pl.multiple_of(s * RPT, 8)
        pltpu.sync_copy(acc_sh.at[pl.ds(off, RPT)],
                        agg_hbm.at[c, pl.ds(off, RPT)])

    out_type = jax.ShapeDtypeStruct((NC, NP, D), jnp.float32)
    scratch = (
        [pltpu.VMEM_SHARED((NP, D), jnp.float32),
         pltpu.VMEM((16, D), jnp.float32)]
        + [pltpu.VMEM((CH,), jnp.int32) for _ in range(8)]
        + [pltpu.VMEM((CH, D), jnp.float32)]
    )
    if with_count:
        out_type = [out_type, jax.ShapeDtypeStruct((NW, NP), jnp.float32)]
        scratch = scratch + [pltpu.VMEM((NP,), jnp.float32)]
    scratch = scratch + [pltpu.SemaphoreType.DMA] * 5

    return pl.kernel(
        body, out_type=out_type, mesh=_mesh, scratch_types=scratch,
        compiler_params=pltpu.CompilerParams(needs_layout_passes=False))


_segsum_cnt = _make_segsum(True)
_segsum = _make_segsum(False)


def _decoder_body(a_hbm, b_hbm, u_hbm, v_hbm, w2_hbm, b2_hbm, out_hbm,
                  w2_v, b2_v, ubuf, vbuf, ar0, ar1, br0, br1, ob0, ob1,
                  gs0, gs1, os0, os1):
    ars = (ar0, ar1)
    brs = (br0, br1)
    obs = (ob0, ob1)
    gss = (gs0, gs1)
    oss = (os0, os1)
    c = lax.axis_index("c")
    s = lax.axis_index("s")
    wid = c * NS + s
    roff = pl.multiple_of(wid * DEC_CPW, 8)
    pltpu.sync_copy(u_hbm.at[pl.ds(roff, DEC_CPW)], ubuf)
    pltpu.sync_copy(v_hbm.at[pl.ds(roff, DEC_CPW)], vbuf)
    pltpu.sync_copy(w2_hbm, w2_v)
    pltpu.sync_copy(b2_hbm, b2_v)
    wregs = [w2_v[pl.ds(j * 16, 16)] for j in range(D // 16)]
    b2 = b2_v[...][0]
    lanes = lax.iota(jnp.int32, 16)

    def issue(g, b):
        pltpu.async_copy(a_hbm.at[ubuf.at[g]], ars[b], gss[b])
        pltpu.async_copy(b_hbm.at[vbuf.at[g]], brs[b], gss[b])

    def wait_gather(b):
        pltpu.make_async_copy(a_hbm.at[ubuf.at[0]], ars[b], gss[b]).wait()
        pltpu.make_async_copy(b_hbm.at[vbuf.at[0]], brs[b], gss[b]).wait()

    def issue_out(g, b):
        off = pl.multiple_of((wid * DEC_CPW + g) * CH, 8)
        pltpu.async_copy(obs[b], out_hbm.at[pl.ds(off, CH)], oss[b])

    def wait_out(b):
        pltpu.make_async_copy(obs[b], out_hbm.at[pl.ds(0, CH)],
                              oss[b]).wait()

    def compute(g, b):
        arows, brows = ars[b], brs[b]
        outb = obs[b]

        def group(gr, ecarry):
            res = jnp.zeros((16,), jnp.float32)
            for i in range(16):
                eidx = gr * 16 + i
                acc = jnp.zeros((16,), jnp.float32)
                for j in range(D // 16):
                    t = (arows[eidx, pl.ds(j * 16, 16)]
                         + brows[eidx, pl.ds(j * 16, 16)])
                    acc = acc + jnp.maximum(t, 0.0) * wregs[j]
                res = jnp.where(lanes == i, jnp.sum(acc) + b2, res)
            outb[pl.ds(gr * 16, 16)] = res
            return ecarry

        lax.fori_loop(0, CH // 16, group, 0)

    issue(0, 0)
    issue(1, 1)
    for b in range(2):
        doff = pl.multiple_of((DEC_ROWS + 2 * wid + b) * CH, 8)
        pltpu.async_copy(obs[b], out_hbm.at[pl.ds(doff, CH)], oss[b])

    def step(m, carry):
        for b in range(2):
            g = m * 2 + b
            wait_gather(b)
            wait_out(b)
            compute(g, b)
            issue_out(g, b)
            issue(g + 2, b)
        return carry

    lax.fori_loop(0, DEC_CPW // 2 - 1, step, 0)
    for b in range(2):
        g = DEC_CPW - 2 + b
        wait_gather(b)
        wait_out(b)
        compute(g, b)
        issue_out(g, b)
    wait_out(0)
    wait_out(1)


_decoder = pl.kernel(
    _decoder_body,
    out_type=jax.ShapeDtypeStruct((DEC_OUT,), jnp.float32),
    mesh=_mesh,
    scratch_types=[
        pltpu.VMEM((D,), jnp.float32),
        pltpu.VMEM((16,), jnp.float32),
        pltpu.VMEM((DEC_CPW, CH), jnp.int32),
        pltpu.VMEM((DEC_CPW, CH), jnp.int32),
        pltpu.VMEM((CH, D), jnp.float32),
        pltpu.VMEM((CH, D), jnp.float32),
        pltpu.VMEM((CH, D), jnp.float32),
        pltpu.VMEM((CH, D), jnp.float32),
        pltpu.VMEM((CH,), jnp.float32),
        pltpu.VMEM((CH,), jnp.float32),
        pltpu.SemaphoreType.DMA,
        pltpu.SemaphoreType.DMA,
        pltpu.SemaphoreType.DMA,
        pltpu.SemaphoreType.DMA,
    ],
    compiler_params=pltpu.CompilerParams(needs_layout_passes=False),
)


def kernel(x, edge_index, pos_edge, neg_edge,
           Wl1, bl1, Wr1, Wl2, bl2, Wr2, Wd1, bd1, Wd2, bd2):
    ei = edge_index.astype(jnp.int32)
    segfill = SEG_PAD - E
    src = jnp.concatenate([ei[0], jnp.zeros((segfill,), jnp.int32)])
    dst = jnp.concatenate([ei[1], jnp.full((segfill,), N, jnp.int32)])
    decfill = DEC_PAD - 2 * E
    zfill = jnp.zeros((decfill,), jnp.int32)
    u_all = jnp.concatenate(
        [pos_edge[0].astype(jnp.int32), neg_edge[0].astype(jnp.int32),
         zfill]).reshape(-1, CH)
    v_all = jnp.concatenate(
        [pos_edge[1].astype(jnp.int32), neg_edge[1].astype(jnp.int32),
         zfill]).reshape(-1, CH)

    y1, r1 = _enc_prep(x, Wl1, Wr1, bl1.reshape(1, D))
    agg1, cnt = _segsum_cnt(y1, src, dst)
    y2, r2, invb = _combine1(agg1, cnt, r1, Wl2, Wr2, bl2.reshape(1, D))

    agg2 = _segsum(y2, src, dst)
    adec, bdec = _combine2(agg2, r2, invb, Wd1[:, :D], Wd1[:, D:],
                           bd1.reshape(1, D))

    w2 = Wd2.reshape(D)
    b2p = jnp.broadcast_to(bd2.reshape(1), (16,))
    out = _decoder(adec, bdec, u_all, v_all, w2, b2p)
    return out[:2 * E]


# speedup vs baseline: 1.4444x; 1.4444x over previous
import functools

import jax
import jax.numpy as jnp
from jax import lax
from jax.experimental import pallas as pl
from jax.experimental.pallas import tpu as pltpu
from jax.experimental.pallas import tpu_sc as plsc

N = 10000
E = 320000
D = 128
NC = 2
NS = 16
NW = NC * NS
CH = 80
RB = 1024
NP = 10240
RPT = NP // NS

_mesh = plsc.VectorSubcoreMesh(
    core_axis_name="c", subcore_axis_name="s", num_cores=NC, num_subcores=NS)


def _enc_prep_body(x_ref, wl_ref, wr_ref, bl_ref, y_ref, r_ref):
    xb = x_ref[...]
    y_ref[...] = jnp.dot(xb, wl_ref[...].T, preferred_element_type=jnp.float32)
    r_ref[...] = (jnp.dot(xb, wr_ref[...].T, preferred_element_type=jnp.float32)
                  + bl_ref[...])


def _inv_count_col(cnt_blk):
    ones = jnp.ones((NW, 1), jnp.float32)
    col = lax.dot_general(cnt_blk, ones, (((0,), (0,)), ((), ())),
                          preferred_element_type=jnp.float32)
    return 1.0 / jnp.maximum(col, 1.0)


def _combine1_body(p_ref, cnt_ref, r1_ref, wl2_ref, wr2_ref, bl2_ref,
                   y2_ref, r2_ref, invb_ref):
    inv = _inv_count_col(cnt_ref[...])
    agg = p_ref[0] + p_ref[1]
    h = jnp.maximum(agg * inv + r1_ref[...], 0.0)
    y2_ref[...] = jnp.dot(h, wl2_ref[...].T, preferred_element_type=jnp.float32)
    r2_ref[...] = (jnp.dot(h, wr2_ref[...].T, preferred_element_type=jnp.float32)
                   + bl2_ref[...])
    invb_ref[...] = jnp.broadcast_to(inv, (inv.shape[0], D))


def _combine2_body(p_ref, r2_ref, invb_ref, w1a_ref, w1b_ref, bd1_ref,
                   a_ref, b_ref):
    agg = p_ref[0] + p_ref[1]
    z = agg * invb_ref[...] + r2_ref[...]
    a_ref[...] = jnp.dot(z, w1a_ref[...].T, preferred_element_type=jnp.float32)
    b_ref[...] = (jnp.dot(z, w1b_ref[...].T, preferred_element_type=jnp.float32)
                  + bd1_ref[...])


def _row_spec(width):
    return pl.BlockSpec((RB, width), lambda i: (i, 0))


def _full_spec(shape):
    return pl.BlockSpec(shape, lambda i: tuple(0 for _ in shape))


_enc_prep = pl.pallas_call(
    _enc_prep_body,
    grid=(pl.cdiv(N, RB),),
    in_specs=[_row_spec(D), _full_spec((D, D)), _full_spec((D, D)),
              _full_spec((1, D))],
    out_specs=[_row_spec(D), _row_spec(D)],
    out_shape=[jax.ShapeDtypeStruct((N, D), jnp.float32),
               jax.ShapeDtypeStruct((N, D), jnp.float32)],
)

_combine1 = pl.pallas_call(
    _combine1_body,
    grid=(pl.cdiv(N, RB),),
    in_specs=[pl.BlockSpec((NC, RB, D), lambda i: (0, i, 0)),
              pl.BlockSpec((NW, RB), lambda i: (0, i)),
              _row_spec(D), _full_spec((D, D)), _full_spec((D, D)),
              _full_spec((1, D))],
    out_specs=[_row_spec(D), _row_spec(D), _row_spec(D)],
    out_shape=[jax.ShapeDtypeStruct((N, D), jnp.float32),
               jax.ShapeDtypeStruct((N, D), jnp.float32),
               jax.ShapeDtypeStruct((N, D), jnp.float32)],
)

_combine2 = pl.pallas_call(
    _combine2_body,
    grid=(pl.cdiv(N, RB),),
    in_specs=[pl.BlockSpec((NC, RB, D), lambda i: (0, i, 0)),
              _row_spec(D), _row_spec(D), _full_spec((D, D)),
              _full_spec((D, D)), _full_spec((1, D))],
    out_specs=[_row_spec(D), _row_spec(D)],
    out_shape=[jax.ShapeDtypeStruct((N, D), jnp.float32),
               jax.ShapeDtypeStruct((N, D), jnp.float32)],
)


def _zero_spmem_slice(acc_sh, zrow_v, s):
    zr = zrow_v.shape[0]

    def zb(i, carry):
        for j in range(D // 16):
            zrow_v[i, pl.ds(j * 16, 16)] = jnp.zeros((16,), jnp.float32)
        return carry

    lax.fori_loop(0, zr, zb, 0)

    def zcp(k, carry):
        off = pl.multiple_of(s * RPT + k * zr, 8)
        pltpu.sync_copy(zrow_v, acc_sh.at[pl.ds(off, zr)])
        return carry

    lax.fori_loop(0, RPT // zr, zcp, 0)


def _make_segsum(with_count):
    epw = E // NW
    cpw = epw // CH

    def body(y_hbm, sd_hbm, *refs):
        if with_count:
            (agg_hbm, cnt_hbm,
             acc_sh, zrow_v, sdbuf, idx_d, rows_v, hist_v, sem) = refs
        else:
            (agg_hbm, acc_sh, zrow_v, sdbuf, idx_d, rows_v, sem) = refs
        c = lax.axis_index("c")
        s = lax.axis_index("s")
        wid = c * NS + s
        _zero_spmem_slice(acc_sh, zrow_v, s)
        if with_count:
            def zh(i, carry):
                hist_v[pl.ds(i * 16, 16)] = jnp.zeros((16,), jnp.float32)
                return carry

            lax.fori_loop(0, NP // 16, zh, 0)
        plsc.subcore_barrier()

        def chunk(k, carry):
            base = pl.multiple_of((wid * cpw + k) * 2 * CH, 8)
            pltpu.sync_copy(sd_hbm.at[pl.ds(base, 2 * CH)], sdbuf)
            gat = pltpu.async_copy(y_hbm.at[sdbuf.at[pl.ds(0, CH)]],
                                   rows_v, sem)
            for t in range(CH // 16):
                idx_d[pl.ds(t * 16, 16)] = sdbuf[pl.ds(CH + t * 16, 16)]
            if with_count:
                for t in range(CH // 16):
                    d16 = idx_d[pl.ds(t * 16, 16)]
                    cnts, lastm = plsc.scan_count(d16)
                    plsc.addupdate_scatter(hist_v, [d16],
                                           cnts.astype(jnp.float32),
                                           mask=lastm)
            gat.wait()
            pltpu.sync_copy(rows_v, acc_sh.at[idx_d], add=True)
            return carry

        lax.fori_loop(0, cpw, chunk, 0)
        if with_count:
            pltpu.sync_copy(hist_v, cnt_hbm.at[wid])
        plsc.subcore_barrier()
        off = pl.multiple_of(s * RPT, 8)
        pltpu.sync_copy(acc_sh.at[pl.ds(off, RPT)],
                        agg_hbm.at[c, pl.ds(off, RPT)])

    out_type = jax.ShapeDtypeStruct((NC, NP, D), jnp.float32)
    scratch = [
        pltpu.VMEM_SHARED((NP, D), jnp.float32),
        pltpu.VMEM((128, D), jnp.float32),
        pltpu.VMEM((2 * CH,), jnp.int32),
        pltpu.VMEM((CH,), jnp.int32),
        pltpu.VMEM((CH, D), jnp.float32),
    ]
    if with_count:
        out_type = [out_type, jax.ShapeDtypeStruct((NW, NP), jnp.float32)]
        scratch = scratch + [pltpu.VMEM((NP,), jnp.float32)]
    scratch = scratch + [pltpu.SemaphoreType.DMA]

    return pl.kernel(
        body, out_type=out_type, mesh=_mesh, scratch_types=scratch,
        compiler_params=pltpu.CompilerParams(needs_layout_passes=False))


_segsum_cnt = _make_segsum(True)
_segsum = _make_segsum(False)


def _decoder_body(a_hbm, b_hbm, uv_hbm, w2_hbm, b2_hbm, out_hbm,
                  w2_v, b2_v, uvbuf, arows, brows, outb, sem):
    ne = 2 * E
    dpw = ne // NW
    cpw = dpw // CH
    c = lax.axis_index("c")
    s = lax.axis_index("s")
    wid = c * NS + s
    pltpu.sync_copy(w2_hbm, w2_v)
    pltpu.sync_copy(b2_hbm, b2_v)
    wregs = [w2_v[pl.ds(j * 16, 16)] for j in range(D // 16)]
    b2 = b2_v[...][0]

    def chunk(k, carry):
        base = pl.multiple_of(wid * dpw + k * CH, 8)
        uvbase = pl.multiple_of((wid * cpw + k) * 2 * CH, 8)
        pltpu.sync_copy(uv_hbm.at[pl.ds(uvbase, 2 * CH)], uvbuf)
        ca = pltpu.async_copy(a_hbm.at[uvbuf.at[pl.ds(0, CH)]], arows, sem)
        cb = pltpu.async_copy(b_hbm.at[uvbuf.at[pl.ds(CH, CH)]], brows, sem)
        ca.wait()
        cb.wait()
        lanes = lax.iota(jnp.int32, 16)

        def group(g, ecarry):
            res = jnp.zeros((16,), jnp.float32)
            for i in range(16):
                eidx = g * 16 + i
                acc = jnp.zeros((16,), jnp.float32)
                for j in range(D // 16):
                    t = (arows[eidx, pl.ds(j * 16, 16)]
                         + brows[eidx, pl.ds(j * 16, 16)])
                    acc = acc + jnp.maximum(t, 0.0) * wregs[j]
                res = jnp.where(lanes == i, jnp.sum(acc) + b2, res)
            outb[pl.ds(g * 16, 16)] = res
            return ecarry

        lax.fori_loop(0, CH // 16, group, 0)
        pltpu.sync_copy(outb, out_hbm.at[pl.ds(base, CH)])
        return carry

    lax.fori_loop(0, dpw // CH, chunk, 0)


_decoder = pl.kernel(
    _decoder_body,
    out_type=jax.ShapeDtypeStruct((2 * E,), jnp.float32),
    mesh=_mesh,
    scratch_types=[
        pltpu.VMEM((D,), jnp.float32),
        pltpu.VMEM((16,), jnp.float32),
        pltpu.VMEM((2 * CH,), jnp.int32),
        pltpu.VMEM((CH, D), jnp.float32),
        pltpu.VMEM((CH, D), jnp.float32),
        pltpu.VMEM((CH,), jnp.float32),
        pltpu.SemaphoreType.DMA,
    ],
    compiler_params=pltpu.CompilerParams(needs_layout_passes=False),
)


def kernel(x, edge_index, pos_edge, neg_edge,
           Wl1, bl1, Wr1, Wl2, bl2, Wr2, Wd1, bd1, Wd2, bd2):
    ei = edge_index.astype(jnp.int32)
    sd = jnp.stack([ei[0].reshape(-1, CH), ei[1].reshape(-1, CH)],
                   axis=1).reshape(-1)
    u_all = jnp.concatenate([pos_edge[0], neg_edge[0]]).astype(jnp.int32)
    v_all = jnp.concatenate([pos_edge[1], neg_edge[1]]).astype(jnp.int32)
    uv = jnp.stack([u_all.reshape(-1, CH), v_all.reshape(-1, CH)],
                   axis=1).reshape(-1)

    y1, r1 = _enc_prep(x, Wl1, Wr1, bl1.reshape(1, D))
    agg1, cnt = _segsum_cnt(y1, sd)
    y2, r2, invb = _combine1(agg1, cnt, r1, Wl2, Wr2, bl2.reshape(1, D))

    agg2 = _segsum(y2, sd)
    adec, bdec = _combine2(agg2, r2, invb, Wd1[:, :D], Wd1[:, D:],
                           bd1.reshape(1, D))

    w2 = Wd2.reshape(D)
    b2p = jnp.broadcast_to(bd2.reshape(1), (16,))
    return _decoder(adec, bdec, uv, w2, b2p)


# speedup vs baseline: 1.6872x; 1.1681x over previous
import functools

import jax
import jax.numpy as jnp
from jax import lax
from jax.experimental import pallas as pl
from jax.experimental.pallas import tpu as pltpu
from jax.experimental.pallas import tpu_sc as plsc

N = 10000
E = 320000
D = 128
NC = 2
NS = 16
NW = NC * NS
CH = 80
RB = 1024
NP = 10240
RPT = NP // NS

_mesh = plsc.VectorSubcoreMesh(
    core_axis_name="c", subcore_axis_name="s", num_cores=NC, num_subcores=NS)


def _enc_prep_body(x_ref, wl_ref, wr_ref, bl_ref, y_ref, r_ref):
    xb = x_ref[...]
    y_ref[...] = jnp.dot(xb, wl_ref[...].T, preferred_element_type=jnp.float32)
    r_ref[...] = (jnp.dot(xb, wr_ref[...].T, preferred_element_type=jnp.float32)
                  + bl_ref[...])


def _inv_count_col(cnt_blk):
    ones = jnp.ones((NW, 1), jnp.float32)
    col = lax.dot_general(cnt_blk, ones, (((0,), (0,)), ((), ())),
                          preferred_element_type=jnp.float32)
    return 1.0 / jnp.maximum(col, 1.0)


def _combine1_body(p_ref, cnt_ref, r1_ref, wl2_ref, wr2_ref, bl2_ref,
                   y2_ref, r2_ref, invb_ref):
    inv = _inv_count_col(cnt_ref[...])
    agg = p_ref[0] + p_ref[1]
    h = jnp.maximum(agg * inv + r1_ref[...], 0.0)
    y2_ref[...] = jnp.dot(h, wl2_ref[...].T, preferred_element_type=jnp.float32)
    r2_ref[...] = (jnp.dot(h, wr2_ref[...].T, preferred_element_type=jnp.float32)
                   + bl2_ref[...])
    invb_ref[...] = jnp.broadcast_to(inv, (inv.shape[0], D))


def _combine2_body(p_ref, r2_ref, invb_ref, w1a_ref, w1b_ref, bd1_ref,
                   a_ref, b_ref):
    agg = p_ref[0] + p_ref[1]
    z = agg * invb_ref[...] + r2_ref[...]
    a_ref[...] = jnp.dot(z, w1a_ref[...].T, preferred_element_type=jnp.float32)
    b_ref[...] = (jnp.dot(z, w1b_ref[...].T, preferred_element_type=jnp.float32)
                  + bd1_ref[...])


def _row_spec(width):
    return pl.BlockSpec((RB, width), lambda i: (i, 0))


def _full_spec(shape):
    return pl.BlockSpec(shape, lambda i: tuple(0 for _ in shape))


_enc_prep = pl.pallas_call(
    _enc_prep_body,
    grid=(pl.cdiv(N, RB),),
    in_specs=[_row_spec(D), _full_spec((D, D)), _full_spec((D, D)),
              _full_spec((1, D))],
    out_specs=[_row_spec(D), _row_spec(D)],
    out_shape=[jax.ShapeDtypeStruct((N, D), jnp.float32),
               jax.ShapeDtypeStruct((N, D), jnp.float32)],
)

_combine1 = pl.pallas_call(
    _combine1_body,
    grid=(pl.cdiv(N, RB),),
    in_specs=[pl.BlockSpec((NC, RB, D), lambda i: (0, i, 0)),
              pl.BlockSpec((NW, RB), lambda i: (0, i)),
              _row_spec(D), _full_spec((D, D)), _full_spec((D, D)),
              _full_spec((1, D))],
    out_specs=[_row_spec(D), _row_spec(D), _row_spec(D)],
    out_shape=[jax.ShapeDtypeStruct((N, D), jnp.float32),
               jax.ShapeDtypeStruct((N, D), jnp.float32),
               jax.ShapeDtypeStruct((N, D), jnp.float32)],
)

_combine2 = pl.pallas_call(
    _combine2_body,
    grid=(pl.cdiv(N, RB),),
    in_specs=[pl.BlockSpec((NC, RB, D), lambda i: (0, i, 0)),
              _row_spec(D), _row_spec(D), _full_spec((D, D)),
              _full_spec((D, D)), _full_spec((1, D))],
    out_specs=[_row_spec(D), _row_spec(D)],
    out_shape=[jax.ShapeDtypeStruct((N, D), jnp.float32),
               jax.ShapeDtypeStruct((N, D), jnp.float32)],
)


def _zero_spmem_slice(acc_sh, zrow_v, s):
    zr = zrow_v.shape[0]

    def zb(i, carry):
        for j in range(D // 16):
            zrow_v[i, pl.ds(j * 16, 16)] = jnp.zeros((16,), jnp.float32)
        return carry

    lax.fori_loop(0, zr, zb, 0)

    def zcp(k, carry):
        off = pl.multiple_of(s * RPT + k * zr, 8)
        pltpu.sync_copy(zrow_v, acc_sh.at[pl.ds(off, zr)])
        return carry

    lax.fori_loop(0, RPT // zr, zcp, 0)


def _make_segsum(with_count):
    epw = E // NW
    cpw = epw // CH

    def body(y_hbm, sd_hbm, *refs):
        if with_count:
            (agg_hbm, cnt_hbm,
             acc_sh, zrow_v, sdbuf, idx_d, rows_v, hist_v, sem) = refs
        else:
            (agg_hbm, acc_sh, zrow_v, sdbuf, idx_d, rows_v, sem) = refs
        c = lax.axis_index("c")
        s = lax.axis_index("s")
        wid = c * NS + s
        _zero_spmem_slice(acc_sh, zrow_v, s)
        if with_count:
            def zh(i, carry):
                hist_v[pl.ds(i * 16, 16)] = jnp.zeros((16,), jnp.float32)
                return carry

            lax.fori_loop(0, NP // 16, zh, 0)
        plsc.subcore_barrier()

        def chunk(k, carry):
            base = pl.multiple_of((wid * cpw + k) * 2 * CH, 8)
            pltpu.sync_copy(sd_hbm.at[pl.ds(base, 2 * CH)], sdbuf)
            gat = pltpu.async_copy(y_hbm.at[sdbuf.at[pl.ds(0, CH)]],
                                   rows_v, sem)
            for t in range(CH // 16):
                idx_d[pl.ds(t * 16, 16)] = sdbuf[pl.ds(CH + t * 16, 16)]
            if with_count:
                for t in range(CH // 16):
                    d16 = idx_d[pl.ds(t * 16, 16)]
                    cnts, lastm = plsc.scan_count(d16)
                    plsc.addupdate_scatter(hist_v, [d16],
                                           cnts.astype(jnp.float32),
                                           mask=lastm)
            gat.wait()
            pltpu.sync_copy(rows_v, acc_sh.at[idx_d], add=True)
            return carry

        lax.fori_loop(0, cpw, chunk, 0)
        if with_count:
            pltpu.sync_copy(hist_v, cnt_hbm.at[wid])
        plsc.subcore_barrier()
        off = pl.multiple_of(s * RPT, 8)
        pltpu.sync_copy(acc_sh.at[pl.ds(off, RPT)],
                        agg_hbm.at[c, pl.ds(off, RPT)])

    out_type = jax.ShapeDtypeStruct((NC, NP, D), jnp.float32)
    scratch = [
        pltpu.VMEM_SHARED((NP, D), jnp.float32),
        pltpu.VMEM((128, D), jnp.float32),
        pltpu.VMEM((2 * CH,), jnp.int32),
        pltpu.VMEM((CH,), jnp.int32),
        pltpu.VMEM((CH, D), jnp.float32),
    ]
    if with_count:
        out_type = [out_type, jax.ShapeDtypeStruct((NW, NP), jnp.float32)]
        scratch = scratch + [pltpu.VMEM((NP,), jnp.float32)]
    scratch = scratch + [pltpu.SemaphoreType.DMA]

    return pl.kernel(
        body, out_type=out_type, mesh=_mesh, scratch_types=scratch,
        compiler_params=pltpu.CompilerParams(needs_layout_passes=False))


_segsum_cnt = _make_segsum(True)
_segsum = _make_segsum(False)


def _decoder_body(a_hbm, b_hbm, uv_hbm, w2_hbm, b2_hbm, out_hbm,
                  w2_v, b2_v, uv0, uv1, ar0, ar1, br0, br1, outb,
                  sem0, sem1):
    ne = 2 * E
    dpw = ne // NW
    cpw = dpw // CH
    uvs = (uv0, uv1)
    ars = (ar0, ar1)
    brs = (br0, br1)
    sems = (sem0, sem1)
    c = lax.axis_index("c")
    s = lax.axis_index("s")
    wid = c * NS + s
    pltpu.sync_copy(w2_hbm, w2_v)
    pltpu.sync_copy(b2_hbm, b2_v)
    wregs = [w2_v[pl.ds(j * 16, 16)] for j in range(D // 16)]
    b2 = b2_v[...][0]
    lanes = lax.iota(jnp.int32, 16)

    def fetch(g, b):
        uvbase = pl.multiple_of((wid * cpw + (g % cpw)) * 2 * CH, 8)
        pltpu.sync_copy(uv_hbm.at[pl.ds(uvbase, 2 * CH)], uvs[b])
        pltpu.async_copy(a_hbm.at[uvs[b].at[pl.ds(0, CH)]], ars[b], sems[b])
        pltpu.async_copy(b_hbm.at[uvs[b].at[pl.ds(CH, CH)]], brs[b], sems[b])

    def wait_gather(b):
        pltpu.make_async_copy(a_hbm.at[uvs[b].at[pl.ds(0, CH)]], ars[b],
                              sems[b]).wait()
        pltpu.make_async_copy(b_hbm.at[uvs[b].at[pl.ds(0, CH)]], brs[b],
                              sems[b]).wait()

    def consume(g, b):
        arows, brows = ars[b], brs[b]
        wait_gather(b)

        def group(gr, ecarry):
            res = jnp.zeros((16,), jnp.float32)
            for i in range(16):
                eidx = gr * 16 + i
                acc = jnp.zeros((16,), jnp.float32)
                for j in range(D // 16):
                    t = (arows[eidx, pl.ds(j * 16, 16)]
                         + brows[eidx, pl.ds(j * 16, 16)])
                    acc = acc + jnp.maximum(t, 0.0) * wregs[j]
                res = jnp.where(lanes == i, jnp.sum(acc) + b2, res)
            outb[pl.ds(gr * 16, 16)] = res
            return ecarry

        lax.fori_loop(0, CH // 16, group, 0)
        base = pl.multiple_of(wid * dpw + g * CH, 8)
        pltpu.sync_copy(outb, out_hbm.at[pl.ds(base, CH)])

    fetch(0, 0)

    def step(m, carry):
        for b in range(2):
            g = m * 2 + b
            fetch(g + 1, 1 - b)
            consume(g, b)
        return carry

    lax.fori_loop(0, cpw // 2, step, 0)
    wait_gather(0)


_decoder = pl.kernel(
    _decoder_body,
    out_type=jax.ShapeDtypeStruct((2 * E,), jnp.float32),
    mesh=_mesh,
    scratch_types=[
        pltpu.VMEM((D,), jnp.float32),
        pltpu.VMEM((16,), jnp.float32),
        pltpu.VMEM((2 * CH,), jnp.int32),
        pltpu.VMEM((2 * CH,), jnp.int32),
        pltpu.VMEM((CH, D), jnp.float32),
        pltpu.VMEM((CH, D), jnp.float32),
        pltpu.VMEM((CH, D), jnp.float32),
        pltpu.VMEM((CH, D), jnp.float32),
        pltpu.VMEM((CH,), jnp.float32),
        pltpu.SemaphoreType.DMA,
        pltpu.SemaphoreType.DMA,
    ],
    compiler_params=pltpu.CompilerParams(needs_layout_passes=False),
)


def kernel(x, edge_index, pos_edge, neg_edge,
           Wl1, bl1, Wr1, Wl2, bl2, Wr2, Wd1, bd1, Wd2, bd2):
    ei = edge_index.astype(jnp.int32)
    sd = jnp.stack([ei[0].reshape(-1, CH), ei[1].reshape(-1, CH)],
                   axis=1).reshape(-1)
    u_all = jnp.concatenate([pos_edge[0], neg_edge[0]]).astype(jnp.int32)
    v_all = jnp.concatenate([pos_edge[1], neg_edge[1]]).astype(jnp.int32)
    uv = jnp.stack([u_all.reshape(-1, CH), v_all.reshape(-1, CH)],
                   axis=1).reshape(-1)

    y1, r1 = _enc_prep(x, Wl1, Wr1, bl1.reshape(1, D))
    agg1, cnt = _segsum_cnt(y1, sd)
    y2, r2, invb = _combine1(agg1, cnt, r1, Wl2, Wr2, bl2.reshape(1, D))

    agg2 = _segsum(y2, sd)
    adec, bdec = _combine2(agg2, r2, invb, Wd1[:, :D], Wd1[:, D:],
                           bd1.reshape(1, D))

    w2 = Wd2.reshape(D)
    b2p = jnp.broadcast_to(bd2.reshape(1), (16,))
    return _decoder(adec, bdec, uv, w2, b2p)


# speedup vs baseline: 2.0409x; 1.2096x over previous
import functools

import jax
import jax.numpy as jnp
from jax import lax
from jax.experimental import pallas as pl
from jax.experimental.pallas import tpu as pltpu
from jax.experimental.pallas import tpu_sc as plsc

N = 10000
E = 320000
D = 128
NC = 2
NS = 16
NW = NC * NS
CH = 80
RB = 1024
NP = 10240
RPT = NP // NS

_mesh = plsc.VectorSubcoreMesh(
    core_axis_name="c", subcore_axis_name="s", num_cores=NC, num_subcores=NS)


def _enc_prep_body(x_ref, wl_ref, wr_ref, bl_ref, y_ref, r_ref):
    xb = x_ref[...]
    y_ref[...] = jnp.dot(xb, wl_ref[...].T, preferred_element_type=jnp.float32)
    r_ref[...] = (jnp.dot(xb, wr_ref[...].T, preferred_element_type=jnp.float32)
                  + bl_ref[...])


def _inv_count_col(cnt_blk):
    ones = jnp.ones((NW, 1), jnp.float32)
    col = lax.dot_general(cnt_blk, ones, (((0,), (0,)), ((), ())),
                          preferred_element_type=jnp.float32)
    return 1.0 / jnp.maximum(col, 1.0)


def _combine1_body(p_ref, cnt_ref, r1_ref, wl2_ref, wr2_ref, bl2_ref,
                   y2_ref, r2_ref, invb_ref):
    inv = _inv_count_col(cnt_ref[...])
    agg = p_ref[0] + p_ref[1]
    h = jnp.maximum(agg * inv + r1_ref[...], 0.0)
    y2_ref[...] = jnp.dot(h, wl2_ref[...].T, preferred_element_type=jnp.float32)
    r2_ref[...] = (jnp.dot(h, wr2_ref[...].T, preferred_element_type=jnp.float32)
                   + bl2_ref[...])
    invb_ref[...] = jnp.broadcast_to(inv, (inv.shape[0], D))


def _combine2_body(p_ref, r2_ref, invb_ref, w1a_ref, w1b_ref, bd1_ref,
                   a_ref, b_ref):
    agg = p_ref[0] + p_ref[1]
    z = agg * invb_ref[...] + r2_ref[...]
    a_ref[...] = jnp.dot(z, w1a_ref[...].T, preferred_element_type=jnp.float32)
    b_ref[...] = (jnp.dot(z, w1b_ref[...].T, preferred_element_type=jnp.float32)
                  + bd1_ref[...])


def _row_spec(width):
    return pl.BlockSpec((RB, width), lambda i: (i, 0))


def _full_spec(shape):
    return pl.BlockSpec(shape, lambda i: tuple(0 for _ in shape))


_enc_prep = pl.pallas_call(
    _enc_prep_body,
    grid=(pl.cdiv(N, RB),),
    in_specs=[_row_spec(D), _full_spec((D, D)), _full_spec((D, D)),
              _full_spec((1, D))],
    out_specs=[_row_spec(D), _row_spec(D)],
    out_shape=[jax.ShapeDtypeStruct((N, D), jnp.float32),
               jax.ShapeDtypeStruct((N, D), jnp.float32)],
)

_combine1 = pl.pallas_call(
    _combine1_body,
    grid=(pl.cdiv(N, RB),),
    in_specs=[pl.BlockSpec((NC, RB, D), lambda i: (0, i, 0)),
              pl.BlockSpec((NW, RB), lambda i: (0, i)),
              _row_spec(D), _full_spec((D, D)), _full_spec((D, D)),
              _full_spec((1, D))],
    out_specs=[_row_spec(D), _row_spec(D), _row_spec(D)],
    out_shape=[jax.ShapeDtypeStruct((N, D), jnp.float32),
               jax.ShapeDtypeStruct((N, D), jnp.float32),
               jax.ShapeDtypeStruct((N, D), jnp.float32)],
)

_combine2 = pl.pallas_call(
    _combine2_body,
    grid=(pl.cdiv(N, RB),),
    in_specs=[pl.BlockSpec((NC, RB, D), lambda i: (0, i, 0)),
              _row_spec(D), _row_spec(D), _full_spec((D, D)),
              _full_spec((D, D)), _full_spec((1, D))],
    out_specs=[_row_spec(D), _row_spec(D)],
    out_shape=[jax.ShapeDtypeStruct((N, D), jnp.float32),
               jax.ShapeDtypeStruct((N, D), jnp.float32)],
)


def _zero_spmem_slice(acc_sh, zrow_v, s):
    zr = zrow_v.shape[0]

    def zb(i, carry):
        for j in range(D // 16):
            zrow_v[i, pl.ds(j * 16, 16)] = jnp.zeros((16,), jnp.float32)
        return carry

    lax.fori_loop(0, zr, zb, 0)

    def zcp(k, carry):
        off = pl.multiple_of(s * RPT + k * zr, 8)
        pltpu.sync_copy(zrow_v, acc_sh.at[pl.ds(off, zr)])
        return carry

    lax.fori_loop(0, RPT // zr, zcp, 0)


def _make_segsum(with_count):
    epw = E // NW
    cpw = epw // CH

    def body(y_hbm, sd_hbm, *refs):
        if with_count:
            (agg_hbm, cnt_hbm, acc_sh, zrow_v, sd0, sd1, id0, id1,
             rows0, rows1, hist_v, sem0, sem1) = refs
        else:
            (agg_hbm, acc_sh, zrow_v, sd0, sd1, id0, id1,
             rows0, rows1, sem0, sem1) = refs
        sds = (sd0, sd1)
        idds = (id0, id1)
        rows = (rows0, rows1)
        sems = (sem0, sem1)
        c = lax.axis_index("c")
        s = lax.axis_index("s")
        wid = c * NS + s
        _zero_spmem_slice(acc_sh, zrow_v, s)
        if with_count:
            def zh(i, carry):
                hist_v[pl.ds(i * 16, 16)] = jnp.zeros((16,), jnp.float32)
                return carry

            lax.fori_loop(0, NP // 16, zh, 0)
        plsc.subcore_barrier()

        def fetch(g, b):
            base = pl.multiple_of((wid * cpw + g) * 2 * CH, 8)
            pltpu.sync_copy(sd_hbm.at[pl.ds(base, 2 * CH)], sds[b])
            pltpu.async_copy(y_hbm.at[sds[b].at[pl.ds(0, CH)]], rows[b],
                             sems[b])

        def consume(g, b):
            sdbuf = sds[b]
            idx_d = idds[b]
            for t in range(CH // 16):
                idx_d[pl.ds(t * 16, 16)] = sdbuf[pl.ds(CH + t * 16, 16)]
            if with_count:
                for t in range(CH // 16):
                    d16 = idx_d[pl.ds(t * 16, 16)]
                    cnts, lastm = plsc.scan_count(d16)
                    plsc.addupdate_scatter(hist_v, [d16],
                                           cnts.astype(jnp.float32),
                                           mask=lastm)
            pltpu.make_async_copy(y_hbm.at[sdbuf.at[pl.ds(0, CH)]], rows[b],
                                  sems[b]).wait()
            pltpu.sync_copy(rows[b], acc_sh.at[idx_d], add=True)

        fetch(0, 0)

        def step(m, carry):
            for b in range(2):
                g = m * 2 + b
                fetch(g + 1, 1 - b)
                consume(g, b)
            return carry

        lax.fori_loop(0, cpw // 2, step, 0)
        consume(cpw - 1, (cpw - 1) % 2)

        if with_count:
            pltpu.sync_copy(hist_v, cnt_hbm.at[wid])
        plsc.subcore_barrier()
        off = pl.multiple_of(s * RPT, 8)
        pltpu.sync_copy(acc_sh.at[pl.ds(off, RPT)],
                        agg_hbm.at[c, pl.ds(off, RPT)])

    out_type = jax.ShapeDtypeStruct((NC, NP, D), jnp.float32)
    scratch = [
        pltpu.VMEM_SHARED((NP, D), jnp.float32),
        pltpu.VMEM((128, D), jnp.float32),
        pltpu.VMEM((2 * CH,), jnp.int32),
        pltpu.VMEM((2 * CH,), jnp.int32),
        pltpu.VMEM((CH,), jnp.int32),
        pltpu.VMEM((CH,), jnp.int32),
        pltpu.VMEM((CH, D), jnp.float32),
        pltpu.VMEM((CH, D), jnp.float32),
    ]
    if with_count:
        out_type = [out_type, jax.ShapeDtypeStruct((NW, NP), jnp.float32)]
        scratch = scratch + [pltpu.VMEM((NP,), jnp.float32)]
    scratch = scratch + [pltpu.SemaphoreType.DMA, pltpu.SemaphoreType.DMA]

    return pl.kernel(
        body, out_type=out_type, mesh=_mesh, scratch_types=scratch,
        compiler_params=pltpu.CompilerParams(needs_layout_passes=False))


_segsum_cnt = _make_segsum(True)
_segsum = _make_segsum(False)


def _decoder_body(a_hbm, b_hbm, uv_hbm, w2_hbm, b2_hbm, out_hbm,
                  w2_v, b2_v, uv0, uv1, ar0, ar1, br0, br1, outb,
                  sem0, sem1):
    ne = 2 * E
    dpw = ne // NW
    cpw = dpw // CH
    uvs = (uv0, uv1)
    ars = (ar0, ar1)
    brs = (br0, br1)
    sems = (sem0, sem1)
    c = lax.axis_index("c")
    s = lax.axis_index("s")
    wid = c * NS + s
    pltpu.sync_copy(w2_hbm, w2_v)
    pltpu.sync_copy(b2_hbm, b2_v)
    wregs = [w2_v[pl.ds(j * 16, 16)] for j in range(D // 16)]
    b2 = b2_v[...][0]
    lanes = lax.iota(jnp.int32, 16)

    def fetch(g, b):
        uvbase = pl.multiple_of((wid * cpw + (g % cpw)) * 2 * CH, 8)
        pltpu.sync_copy(uv_hbm.at[pl.ds(uvbase, 2 * CH)], uvs[b])
        pltpu.async_copy(a_hbm.at[uvs[b].at[pl.ds(0, CH)]], ars[b], sems[b])
        pltpu.async_copy(b_hbm.at[uvs[b].at[pl.ds(CH, CH)]], brs[b], sems[b])

    def wait_gather(b):
        pltpu.make_async_copy(a_hbm.at[uvs[b].at[pl.ds(0, CH)]], ars[b],
                              sems[b]).wait()
        pltpu.make_async_copy(b_hbm.at[uvs[b].at[pl.ds(0, CH)]], brs[b],
                              sems[b]).wait()

    def consume(g, b):
        arows, brows = ars[b], brs[b]
        wait_gather(b)

        def group(gr, ecarry):
            res = jnp.zeros((16,), jnp.float32)
            for i in range(16):
                eidx = gr * 16 + i
                acc = jnp.zeros((16,), jnp.float32)
                for j in range(D // 16):
                    t = (arows[eidx, pl.ds(j * 16, 16)]
                         + brows[eidx, pl.ds(j * 16, 16)])
                    acc = acc + jnp.maximum(t, 0.0) * wregs[j]
                res = jnp.where(lanes == i, jnp.sum(acc) + b2, res)
            outb[pl.ds(gr * 16, 16)] = res
            return ecarry

        lax.fori_loop(0, CH // 16, group, 0)
        base = pl.multiple_of(wid * dpw + g * CH, 8)
        pltpu.sync_copy(outb, out_hbm.at[pl.ds(base, CH)])

    fetch(0, 0)

    def step(m, carry):
        for b in range(2):
            g = m * 2 + b
            fetch(g + 1, 1 - b)
            consume(g, b)
        return carry

    lax.fori_loop(0, cpw // 2, step, 0)
    wait_gather(0)


_decoder = pl.kernel(
    _decoder_body,
    out_type=jax.ShapeDtypeStruct((2 * E,), jnp.float32),
    mesh=_mesh,
    scratch_types=[
        pltpu.VMEM((D,), jnp.float32),
        pltpu.VMEM((16,), jnp.float32),
        pltpu.VMEM((2 * CH,), jnp.int32),
        pltpu.VMEM((2 * CH,), jnp.int32),
        pltpu.VMEM((CH, D), jnp.float32),
        pltpu.VMEM((CH, D), jnp.float32),
        pltpu.VMEM((CH, D), jnp.float32),
        pltpu.VMEM((CH, D), jnp.float32),
        pltpu.VMEM((CH,), jnp.float32),
        pltpu.SemaphoreType.DMA,
        pltpu.SemaphoreType.DMA,
    ],
    compiler_params=pltpu.CompilerParams(needs_layout_passes=False),
)


def kernel(x, edge_index, pos_edge, neg_edge,
           Wl1, bl1, Wr1, Wl2, bl2, Wr2, Wd1, bd1, Wd2, bd2):
    ei = edge_index.astype(jnp.int32)
    sd = jnp.stack([ei[0].reshape(-1, CH), ei[1].reshape(-1, CH)],
                   axis=1).reshape(-1)
    u_all = jnp.concatenate([pos_edge[0], neg_edge[0]]).astype(jnp.int32)
    v_all = jnp.concatenate([pos_edge[1], neg_edge[1]]).astype(jnp.int32)
    uv = jnp.stack([u_all.reshape(-1, CH), v_all.reshape(-1, CH)],
                   axis=1).reshape(-1)

    y1, r1 = _enc_prep(x, Wl1, Wr1, bl1.reshape(1, D))
    agg1, cnt = _segsum_cnt(y1, sd)
    y2, r2, invb = _combine1(agg1, cnt, r1, Wl2, Wr2, bl2.reshape(1, D))

    agg2 = _segsum(y2, sd)
    adec, bdec = _combine2(agg2, r2, invb, Wd1[:, :D], Wd1[:, D:],
                           bd1.reshape(1, D))

    w2 = Wd2.reshape(D)
    b2p = jnp.broadcast_to(bd2.reshape(1), (16,))
    return _decoder(adec, bdec, uv, w2, b2p)
